# baseline (speedup 1.0000x reference)
import functools

import jax
import jax.numpy as jnp
from jax import lax
from jax.experimental import pallas as pl
from jax.experimental.pallas import tpu as pltpu
from jax.experimental.pallas import tpu_sc as plsc

NC = 2
NS = 16
NW = NC * NS
CHUNK = 128


def _make_gather(V, D, B):
  b_per_w = B // NW
  n_chunks = b_per_w // CHUNK
  mesh = plsc.VectorSubcoreMesh(
      core_axis_name="c", subcore_axis_name="s", num_cores=NC,
      num_subcores=NS)

  @functools.partial(
      pl.kernel,
      mesh=mesh,
      out_type=jax.ShapeDtypeStruct((D * B,), jnp.float32),
      scratch_types=[
          pltpu.VMEM((n_chunks, CHUNK), jnp.int32),
          pltpu.VMEM((D, b_per_w), jnp.float32),
          pltpu.SemaphoreType.DMA,
      ],
      compiler_params=pltpu.CompilerParams(use_tc_tiling_on_sc=False),
  )
  def gather_kernel(table_hbm, idx_hbm, out_hbm, idx_v, rows_v, sem):
    wid = lax.axis_index("s") * NC + lax.axis_index("c")
    pltpu.sync_copy(idx_hbm.at[pl.ds(wid * n_chunks, n_chunks)], idx_v)
    copies = []
    for d in range(D):
      for j in range(n_chunks):
        copies.append(
            pltpu.async_copy(
                table_hbm.at[d].at[idx_v.at[j]],
                rows_v.at[d, pl.ds(j * CHUNK, CHUNK)],
                sem,
            ))
    for c in copies:
      c.wait()
    for d in range(D):
      pltpu.sync_copy(rows_v.at[d],
                      out_hbm.at[pl.ds(d * B + wid * b_per_w, b_per_w)])

  return gather_kernel


def kernel(indices, table):
  V, D = table.shape
  (B,) = indices.shape
  idx2d = indices.astype(jnp.int32).reshape(B // CHUNK, CHUNK)
  out_flat = _make_gather(V, D, B)(table.T, idx2d)
  return out_flat.reshape(D, B).T


# speedup vs baseline: 17.6036x; 17.6036x over previous
import functools

import jax
import jax.numpy as jnp
from jax import lax
from jax.experimental import pallas as pl
from jax.experimental.pallas import tpu as pltpu
from jax.experimental.pallas import tpu_sc as plsc

NC = 2
NS = 16
NW = NC * NS
CHUNK = 128
W = 4096
L = 16


def _make_detile(V, D, main_cols, tail_lo, rowlen):
  n_unit = main_cols // W + 1
  slots = 31
  tail_cols = V - tail_lo
  flat_sz = D * rowlen
  mesh = plsc.VectorSubcoreMesh(
      core_axis_name="c", subcore_axis_name="s", num_cores=NC,
      num_subcores=NS)

  @functools.partial(
      pl.kernel,
      mesh=mesh,
      out_type=jax.ShapeDtypeStruct((flat_sz,), jnp.float32),
      scratch_types=[
          pltpu.VMEM((2, 8, W), jnp.float32),
          pltpu.VMEM((D, tail_cols), jnp.float32),
          pltpu.SemaphoreType.DMA,
          pltpu.SemaphoreType.DMA,
          pltpu.SemaphoreType.DMA,
      ],
  )
  def detile_kernel(table_hbm, tail_hbm, flat_hbm, win_v, tail_v, sem0,
                    sem1, semw):
    wid = lax.axis_index("s") * NC + lax.axis_index("c")
    r = wid % 4
    g = wid // 4
    base = 31 * g - jnp.maximum(g - 5, 0)
    my_n = jnp.where(g < 5, 31, 30)
    r8 = pl.multiple_of(8 * r, 8)
    sems = [sem0, sem1]

    def col0(s):
      j = base + jnp.minimum(s, my_n - 1)
      return pl.multiple_of(
          jnp.where(j == n_unit - 1, main_cols - W, j * W), 512)

    reads = [None, None]
    reads[0] = pltpu.async_copy(
        table_hbm.at[pl.ds(r8, 8), pl.ds(col0(0), W)], win_v.at[0],
        sems[0])
    for s in range(slots):
      if s + 1 < slots:
        reads[(s + 1) % 2] = pltpu.async_copy(
            table_hbm.at[pl.ds(r8, 8), pl.ds(col0(s + 1), W)],
            win_v.at[(s + 1) % 2], sems[(s + 1) % 2])
      reads[s % 2].wait()
      c0 = col0(s)
      writes = []
      for sub in range(8):
        writes.append(
            pltpu.async_copy(
                win_v.at[s % 2, sub],
                flat_hbm.at[pl.ds((r8 + sub) * rowlen + c0, W)], semw))
      for w_ in writes:
        w_.wait()
    pltpu.sync_copy(tail_hbm, tail_v)
    for sub in range(8):
      pltpu.sync_copy(
          tail_v.at[r8 + sub],
          flat_hbm.at[pl.ds((r8 + sub) * rowlen + main_cols, tail_cols)])

  return detile_kernel


def _make_gather(V, D, B, main_cols, tail_lo, rowlen):
  b_per_w = B // NW
  n_chunks = b_per_w // CHUNK
  mesh = plsc.VectorSubcoreMesh(
      core_axis_name="c", subcore_axis_name="s", num_cores=NC,
      num_subcores=NS)

  @functools.partial(
      pl.kernel,
      mesh=mesh,
      out_type=jax.ShapeDtypeStruct((D * B,), jnp.float32),
      scratch_types=[
          pltpu.VMEM((n_chunks, CHUNK), jnp.int32),
          pltpu.VMEM((n_chunks, CHUNK), jnp.int32),
          pltpu.VMEM((D, b_per_w), jnp.float32),
          pltpu.SemaphoreType.DMA,
      ],
  )
  def gather_kernel(flat_hbm, idx_hbm, out_hbm, idx_v, adj_v, rows_v, sem):
    wid = lax.axis_index("s") * NC + lax.axis_index("c")
    pltpu.sync_copy(idx_hbm.at[pl.ds(wid * n_chunks, n_chunks)], idx_v)
    for j in range(n_chunks):
      for q in range(CHUNK // L):
        v = idx_v[j, pl.ds(q * L, L)]
        adj_v[j, pl.ds(q * L, L)] = jnp.where(
            v >= main_cols, v + (main_cols - tail_lo), v)
    copies = []
    for d in range(D):
      row = flat_hbm.at[pl.ds(d * rowlen, rowlen)]
      for j in range(n_chunks):
        copies.append(
            pltpu.async_copy(
                row.at[adj_v.at[j]],
                rows_v.at[d, pl.ds(j * CHUNK, CHUNK)], sem))
    for c in copies:
      c.wait()
    for d in range(D):
      pltpu.sync_copy(rows_v.at[d],
                      out_hbm.at[pl.ds(d * B + wid * b_per_w, b_per_w)])

  return gather_kernel


def kernel(indices, table):
  V, D = table.shape
  (B,) = indices.shape
  main_cols = (V // CHUNK) * CHUNK
  tail_lo = V - CHUNK
  rowlen = main_cols + CHUNK
  table_t = table.T
  tail_t = table_t[:, tail_lo:]
  idx2d = indices.astype(jnp.int32).reshape(B // CHUNK, CHUNK)
  flat = _make_detile(V, D, main_cols, tail_lo, rowlen)(table_t, tail_t)
  out_flat = _make_gather(V, D, B, main_cols, tail_lo, rowlen)(flat, idx2d)
  return out_flat.reshape(D, B).T
